# baseline (speedup 1.0000x reference)
import jax
import jax.numpy as jnp
from jax import lax
from jax.experimental import pallas as pl
from jax.experimental.pallas import tpu as pltpu
from jax.experimental.pallas import tpu_sc as plsc

N = 10000
NPAD = 10240
E = 320000
B = 128
EPAD = 327680
NCHUNK = EPAD // B
NC = 2
NS = 16
CPT = NCHUNK // NS
HALF = N // 2
TRASH = HALF
ACCROWS = 5008


def _make_seg():
  def body(h_hbm, pc_hbm, w_hbm, z_hbm, out_hbm,
           pc_v, w_v, r0, r1, r2, x0, x1, x2, c0, c1, c2, acc,
           g0, g1, g2, s0, s1, s2):
    rows = [r0, r1, r2]
    idx = [x0, x1, x2]
    cidx = [c0, c1, c2]
    gsem = [g0, g1, g2]
    ssem = [s0, s1, s2]
    c = lax.axis_index("c")
    s = lax.axis_index("s")
    base = c * HALF
    pltpu.sync_copy(z_hbm.at[pl.ds(s * 312, 312)],
                    acc.at[pl.ds(s * 312, 312)])

    @pl.when(s == NS - 1)
    def _():
      pltpu.sync_copy(z_hbm.at[pl.ds(4992, 16)], acc.at[pl.ds(4992, 16)])
    cb = s * CPT
    pltpu.sync_copy(pc_hbm.at[pl.ds(cb, CPT)], pc_v)
    pltpu.sync_copy(w_hbm.at[pl.ds(cb * B, CPT * B)], w_v)
    plsc.subcore_barrier()

    def unpack_cols(k, b):
      def ugroup(g, carry2):
        sl16 = pl.ds(g * 16, 16)
        pcv = pc_v[k, sl16]
        riv = lax.shift_right_logical(pcv, 14) - base
        inb = (riv >= 0) & (riv < HALF)
        cidx[b][sl16] = jnp.where(inb, pcv & 16383, 0)
        return carry2

      lax.fori_loop(0, B // 16, ugroup, 0)

    def scale(k, b):
      def group(g, carry2):
        sl16 = pl.ds(g * 16, 16)
        riv = lax.shift_right_logical(pc_v[k, sl16], 14) - base
        inb = (riv >= 0) & (riv < HALF)
        idx[b][sl16] = jnp.where(inb, riv, TRASH)
        wg = w_v[pl.ds(k * B + g * 16, 16)]
        for j in range(16):
          wb = lax.gather(
              wg, jnp.full((16, 1), j, jnp.int32),
              lax.GatherDimensionNumbers(offset_dims=(),
                                         collapsed_slice_dims=(0,),
                                         start_index_map=(0,)),
              slice_sizes=(1,),
              mode=lax.GatherScatterMode.PROMISE_IN_BOUNDS)
          e = g * 16 + j
          for cb16 in range(8):
            sl = pl.ds(cb16 * 16, 16)
            rows[b][e, sl] = rows[b][e, sl] * wb
        return carry2

      lax.fori_loop(0, B // 16, group, 0)

    def gwait(b):
      pltpu.make_async_copy(h_hbm.at[cidx[b]], rows[b], gsem[b]).wait()

    def gissue(k, b):
      unpack_cols(k, b)
      pltpu.async_copy(h_hbm.at[cidx[b]], rows[b], gsem[b])

    def swait(b):
      pltpu.make_async_copy(rows[b], acc.at[idx[b]], ssem[b]).wait()

    def sissue(b):
      pltpu.async_copy(rows[b], acc.at[idx[b]], ssem[b], add=True)

    def halfstep(k, b):
      gwait(b)

      @pl.when(k >= 1)
      def _():
        swait((b + 2) % 3)

      @pl.when(k + 2 < CPT)
      def _():
        gissue(k + 2, (b + 2) % 3)

      scale(k, b)
      sissue(b)

    gissue(0, 0)
    gissue(1, 1)

    def outer(i, carry):
      k0 = 3 * i
      for b in range(3):
        halfstep(k0 + b, b)
      return carry

    lax.fori_loop(0, (CPT - 1) // 3, outer, 0)
    halfstep(CPT - 1, (CPT - 1) % 3)
    swait((CPT - 1) % 3)
    plsc.subcore_barrier()
    pltpu.sync_copy(acc.at[pl.ds(s * 312, 312)],
                    out_hbm.at[pl.ds(base + s * 312, 312)])

    @pl.when(s == NS - 1)
    def _():
      pltpu.sync_copy(acc.at[pl.ds(4992, 8)],
                      out_hbm.at[pl.ds(base + 4992, 8)])

  return pl.kernel(
      body,
      out_type=jax.ShapeDtypeStruct((N, 128), jnp.float32),
      mesh=plsc.VectorSubcoreMesh(core_axis_name="c", subcore_axis_name="s"),
      scratch_types=[
          pltpu.VMEM((CPT, B), jnp.int32),
          pltpu.VMEM((CPT * B,), jnp.float32),
          pltpu.VMEM((B, 128), jnp.float32),
          pltpu.VMEM((B, 128), jnp.float32),
          pltpu.VMEM((B, 128), jnp.float32),
          pltpu.VMEM((B,), jnp.int32),
          pltpu.VMEM((B,), jnp.int32),
          pltpu.VMEM((B,), jnp.int32),
          pltpu.VMEM((B,), jnp.int32),
          pltpu.VMEM((B,), jnp.int32),
          pltpu.VMEM((B,), jnp.int32),
          pltpu.VMEM_SHARED((ACCROWS, 128), jnp.float32),
          pltpu.SemaphoreType.DMA,
          pltpu.SemaphoreType.DMA,
          pltpu.SemaphoreType.DMA,
          pltpu.SemaphoreType.DMA,
          pltpu.SemaphoreType.DMA,
          pltpu.SemaphoreType.DMA,
      ],
      name="seg_sum",
  )


_seg = _make_seg()

R = 1000


def _ln(h, g, b):
  m = jnp.mean(h, axis=-1, keepdims=True)
  v = jnp.mean((h - m) ** 2, axis=-1, keepdims=True)
  return (h - m) * lax.rsqrt(v + 1e-5) * g + b


def _mlp1_body(x, agg, eps, w1a, b1a, w1b, b1b, g1, be1, out_a, out_b):
  z = (1.0 + eps[0, 0]) * x[...] + agg[...]
  a = jnp.maximum(
      jnp.dot(z, w1a[...], preferred_element_type=jnp.float32) + b1a[...], 0.0)
  h = jnp.dot(a, w1b[...], preferred_element_type=jnp.float32) + b1b[...]
  h = _ln(jnp.maximum(h, 0.0), g1[...], be1[...])
  out_a[...] = h[:, :128]
  out_b[...] = h[:, 128:]


def _mlp2_body(x, h1a, h1b, agga, aggb, eps, w2a, b2a, w2b, b2b, g2, be2,
               wr, br, wf, bf, out):
  h1c = jnp.concatenate([h1a[...], h1b[...]], axis=-1)
  agg = jnp.concatenate([agga[...], aggb[...]], axis=-1)
  z = (1.0 + eps[0, 0]) * h1c + agg
  a = jnp.maximum(
      jnp.dot(z, w2a[...], preferred_element_type=jnp.float32) + b2a[...], 0.0)
  h2 = jnp.dot(a, w2b[...], preferred_element_type=jnp.float32) + b2b[...]
  h2 = _ln(jnp.maximum(h2, 0.0), g2[...], be2[...])
  h = h2 + jnp.dot(x[...], wr[...], preferred_element_type=jnp.float32) + br[...]
  out[...] = jnp.maximum(
      jnp.dot(h, wf[...], preferred_element_type=jnp.float32) + bf[...], 0.0)


def _full(shape):
  return pl.BlockSpec(shape, lambda i: tuple(0 for _ in shape))


_mlp1 = pl.pallas_call(
    _mlp1_body,
    grid=(N // R,),
    in_specs=[
        pl.BlockSpec((R, 128), lambda i: (i, 0)),
        pl.BlockSpec((R, 128), lambda i: (i, 0)),
        _full((1, 1)),
        _full((128, 256)),
        _full((1, 256)),
        _full((256, 256)),
        _full((1, 256)),
        _full((1, 256)),
        _full((1, 256)),
    ],
    out_specs=[pl.BlockSpec((R, 128), lambda i: (i, 0)),
               pl.BlockSpec((R, 128), lambda i: (i, 0))],
    out_shape=[jax.ShapeDtypeStruct((N, 128), jnp.float32),
               jax.ShapeDtypeStruct((N, 128), jnp.float32)],
)

_mlp2 = pl.pallas_call(
    _mlp2_body,
    grid=(N // R,),
    in_specs=[
        pl.BlockSpec((R, 128), lambda i: (i, 0)),
        pl.BlockSpec((R, 128), lambda i: (i, 0)),
        pl.BlockSpec((R, 128), lambda i: (i, 0)),
        pl.BlockSpec((R, 128), lambda i: (i, 0)),
        pl.BlockSpec((R, 128), lambda i: (i, 0)),
        _full((1, 1)),
        _full((256, 128)),
        _full((1, 128)),
        _full((128, 128)),
        _full((1, 128)),
        _full((1, 128)),
        _full((1, 128)),
        _full((128, 128)),
        _full((1, 128)),
        _full((128, 128)),
        _full((1, 128)),
    ],
    out_specs=pl.BlockSpec((R, 128), lambda i: (i, 0)),
    out_shape=jax.ShapeDtypeStruct((N, 128), jnp.float32),
)


def _stream_fwd(x, edge_index, edge_weight, p, zeros):
  pad = EPAD - E
  col = jnp.pad(edge_index[1].astype(jnp.int32), (0, pad))
  row = jnp.pad(edge_index[0].astype(jnp.int32), (0, pad))
  pc = ((row << 14) | col).reshape(NCHUNK, B)
  w = jnp.pad(edge_weight, (0, pad))

  agg1 = _seg(x, pc, w, zeros)
  h1a, h1b = _mlp1(x, agg1, p["eps1"].reshape(1, 1),
                   p["W1a"].T, p["b1a"].reshape(1, -1),
                   p["W1b"].T, p["b1b"].reshape(1, -1),
                   p["g1"].reshape(1, -1), p["be1"].reshape(1, -1))
  agg2a = _seg(h1a, pc, w, zeros)
  h1b2, _ = lax.optimization_barrier((h1b, agg2a))
  agg2b = _seg(h1b2, pc, w, zeros)
  return _mlp2(x, h1a, h1b, agg2a, agg2b, p["eps2"].reshape(1, 1),
               p["W2a"].T, p["b2a"].reshape(1, -1),
               p["W2b"].T, p["b2b"].reshape(1, -1),
               p["g2"].reshape(1, -1), p["be2"].reshape(1, -1),
               p["Wr"].T, p["br"].reshape(1, -1),
               p["Wf"].T, p["bf"].reshape(1, -1))


@jax.jit
def kernel(drug_sim_feat, drug_edge_index, drug_edge_weight,
           disease_sim_feat, dis_edge_index, dis_edge_weight, params):
  zeros = jnp.zeros((NPAD, 128), jnp.float32)
  drug = _stream_fwd(drug_sim_feat, drug_edge_index, drug_edge_weight,
                     params["drug"], zeros)
  x2, _ = lax.optimization_barrier((disease_sim_feat, drug))
  disease = _stream_fwd(x2, dis_edge_index, dis_edge_weight,
                        params["disease"], zeros)
  return (drug, disease)


# speedup vs baseline: 15.3884x; 15.3884x over previous
import jax
import jax.numpy as jnp
from jax import lax
from jax.experimental import pallas as pl
from jax.experimental.pallas import tpu as pltpu
from jax.experimental.pallas import tpu_sc as plsc

N = 10000
NPAD = 10240
E = 320000
B = 128
EPAD = 327680
NCHUNK = EPAD // B
NC = 2
NS = 16
CPT = NCHUNK // NS
HALF = N // 2
TRASH = HALF
ACCROWS = 5008


def _make_seg():
  def body(h_hbm, pc_hbm, w_hbm, z_hbm, out_hbm,
           pc_v, w_v, r0, r1, r2, x0, x1, x2, c0, c1, c2, acc,
           g0, g1, g2, s0, s1, s2):
    rows = [r0, r1, r2]
    idx = [x0, x1, x2]
    cidx = [c0, c1, c2]
    gsem = [g0, g1, g2]
    ssem = [s0, s1, s2]
    c = lax.axis_index("c")
    s = lax.axis_index("s")
    base = c * HALF
    pltpu.sync_copy(z_hbm.at[pl.ds(s * 312, 312)],
                    acc.at[pl.ds(s * 312, 312)])

    @pl.when(s == NS - 1)
    def _():
      pltpu.sync_copy(z_hbm.at[pl.ds(4992, 16)], acc.at[pl.ds(4992, 16)])
    cb = s * CPT
    pltpu.sync_copy(pc_hbm.at[pl.ds(cb, CPT)], pc_v)
    pltpu.sync_copy(w_hbm.at[pl.ds(cb * B, CPT * B)], w_v)
    plsc.subcore_barrier()

    def unpack_cols(k, b):
      def ugroup(g, carry2):
        sl16 = pl.ds(g * 16, 16)
        cidx[b][sl16] = pc_v[k, sl16] & 16383
        return carry2

      lax.fori_loop(0, B // 16, ugroup, 0)

    def scale(k, b):
      def group(g, carry2):
        sl16 = pl.ds(g * 16, 16)
        riv = lax.shift_right_logical(pc_v[k, sl16], 14) - base
        inb = (riv >= 0) & (riv < HALF)
        idx[b][sl16] = jnp.where(inb, riv, TRASH)
        wg = w_v[pl.ds(k * B + g * 16, 16)]
        for j in range(16):
          wb = lax.gather(
              wg, jnp.full((16, 1), j, jnp.int32),
              lax.GatherDimensionNumbers(offset_dims=(),
                                         collapsed_slice_dims=(0,),
                                         start_index_map=(0,)),
              slice_sizes=(1,),
              mode=lax.GatherScatterMode.PROMISE_IN_BOUNDS)
          e = g * 16 + j
          for cb16 in range(8):
            sl = pl.ds(cb16 * 16, 16)
            rows[b][e, sl] = rows[b][e, sl] * wb
        return carry2

      lax.fori_loop(0, B // 16, group, 0)

    def gwait(b):
      pltpu.make_async_copy(h_hbm.at[cidx[b]], rows[b], gsem[b]).wait()

    def gissue(k, b):
      unpack_cols(k, b)
      pltpu.async_copy(h_hbm.at[cidx[b]], rows[b], gsem[b])

    def swait(b):
      pltpu.make_async_copy(rows[b], acc.at[idx[b]], ssem[b]).wait()

    def sissue(b):
      pltpu.async_copy(rows[b], acc.at[idx[b]], ssem[b], add=True)

    def halfstep(k, b):
      gwait(b)

      @pl.when(k >= 1)
      def _():
        swait((b + 2) % 3)

      @pl.when(k + 2 < CPT)
      def _():
        gissue(k + 2, (b + 2) % 3)

      scale(k, b)
      sissue(b)

    gissue(0, 0)
    gissue(1, 1)

    def outer(i, carry):
      k0 = 3 * i
      for b in range(3):
        halfstep(k0 + b, b)
      return carry

    lax.fori_loop(0, (CPT - 1) // 3, outer, 0)
    halfstep(CPT - 1, (CPT - 1) % 3)
    swait((CPT - 1) % 3)
    plsc.subcore_barrier()
    pltpu.sync_copy(acc.at[pl.ds(s * 312, 312)],
                    out_hbm.at[pl.ds(base + s * 312, 312)])

    @pl.when(s == NS - 1)
    def _():
      pltpu.sync_copy(acc.at[pl.ds(4992, 8)],
                      out_hbm.at[pl.ds(base + 4992, 8)])

  return pl.kernel(
      body,
      out_type=jax.ShapeDtypeStruct((N, 128), jnp.float32),
      mesh=plsc.VectorSubcoreMesh(core_axis_name="c", subcore_axis_name="s"),
      scratch_types=[
          pltpu.VMEM((CPT, B), jnp.int32),
          pltpu.VMEM((CPT * B,), jnp.float32),
          pltpu.VMEM((B, 128), jnp.float32),
          pltpu.VMEM((B, 128), jnp.float32),
          pltpu.VMEM((B, 128), jnp.float32),
          pltpu.VMEM((B,), jnp.int32),
          pltpu.VMEM((B,), jnp.int32),
          pltpu.VMEM((B,), jnp.int32),
          pltpu.VMEM((B,), jnp.int32),
          pltpu.VMEM((B,), jnp.int32),
          pltpu.VMEM((B,), jnp.int32),
          pltpu.VMEM_SHARED((ACCROWS, 128), jnp.float32),
          pltpu.SemaphoreType.DMA,
          pltpu.SemaphoreType.DMA,
          pltpu.SemaphoreType.DMA,
          pltpu.SemaphoreType.DMA,
          pltpu.SemaphoreType.DMA,
          pltpu.SemaphoreType.DMA,
      ],
      name="seg_sum",
  )


_seg = _make_seg()

R = 1000


def _ln(h, g, b):
  m = jnp.mean(h, axis=-1, keepdims=True)
  v = jnp.mean((h - m) ** 2, axis=-1, keepdims=True)
  return (h - m) * lax.rsqrt(v + 1e-5) * g + b


def _mlp1_body(x, agg, eps, w1a, b1a, w1b, b1b, g1, be1, out_a, out_b):
  z = (1.0 + eps[0, 0]) * x[...] + agg[...]
  a = jnp.maximum(
      jnp.dot(z, w1a[...], preferred_element_type=jnp.float32) + b1a[...], 0.0)
  h = jnp.dot(a, w1b[...], preferred_element_type=jnp.float32) + b1b[...]
  h = _ln(jnp.maximum(h, 0.0), g1[...], be1[...])
  out_a[...] = h[:, :128]
  out_b[...] = h[:, 128:]


def _mlp2_body(x, h1a, h1b, agga, aggb, eps, w2a, b2a, w2b, b2b, g2, be2,
               wr, br, wf, bf, out):
  h1c = jnp.concatenate([h1a[...], h1b[...]], axis=-1)
  agg = jnp.concatenate([agga[...], aggb[...]], axis=-1)
  z = (1.0 + eps[0, 0]) * h1c + agg
  a = jnp.maximum(
      jnp.dot(z, w2a[...], preferred_element_type=jnp.float32) + b2a[...], 0.0)
  h2 = jnp.dot(a, w2b[...], preferred_element_type=jnp.float32) + b2b[...]
  h2 = _ln(jnp.maximum(h2, 0.0), g2[...], be2[...])
  h = h2 + jnp.dot(x[...], wr[...], preferred_element_type=jnp.float32) + br[...]
  out[...] = jnp.maximum(
      jnp.dot(h, wf[...], preferred_element_type=jnp.float32) + bf[...], 0.0)


def _full(shape):
  return pl.BlockSpec(shape, lambda i: tuple(0 for _ in shape))


_mlp1 = pl.pallas_call(
    _mlp1_body,
    grid=(N // R,),
    in_specs=[
        pl.BlockSpec((R, 128), lambda i: (i, 0)),
        pl.BlockSpec((R, 128), lambda i: (i, 0)),
        _full((1, 1)),
        _full((128, 256)),
        _full((1, 256)),
        _full((256, 256)),
        _full((1, 256)),
        _full((1, 256)),
        _full((1, 256)),
    ],
    out_specs=[pl.BlockSpec((R, 128), lambda i: (i, 0)),
               pl.BlockSpec((R, 128), lambda i: (i, 0))],
    out_shape=[jax.ShapeDtypeStruct((N, 128), jnp.float32),
               jax.ShapeDtypeStruct((N, 128), jnp.float32)],
)

_mlp2 = pl.pallas_call(
    _mlp2_body,
    grid=(N // R,),
    in_specs=[
        pl.BlockSpec((R, 128), lambda i: (i, 0)),
        pl.BlockSpec((R, 128), lambda i: (i, 0)),
        pl.BlockSpec((R, 128), lambda i: (i, 0)),
        pl.BlockSpec((R, 128), lambda i: (i, 0)),
        pl.BlockSpec((R, 128), lambda i: (i, 0)),
        _full((1, 1)),
        _full((256, 128)),
        _full((1, 128)),
        _full((128, 128)),
        _full((1, 128)),
        _full((1, 128)),
        _full((1, 128)),
        _full((128, 128)),
        _full((1, 128)),
        _full((128, 128)),
        _full((1, 128)),
    ],
    out_specs=pl.BlockSpec((R, 128), lambda i: (i, 0)),
    out_shape=jax.ShapeDtypeStruct((N, 128), jnp.float32),
)


def _stream_fwd(x, edge_index, edge_weight, p, zeros):
  pad = EPAD - E
  col = jnp.pad(edge_index[1].astype(jnp.int32), (0, pad))
  row = jnp.pad(edge_index[0].astype(jnp.int32), (0, pad))
  pc = ((row << 14) | col).reshape(NCHUNK, B)
  w = jnp.pad(edge_weight, (0, pad))

  agg1 = _seg(x, pc, w, zeros)
  h1a, h1b = _mlp1(x, agg1, p["eps1"].reshape(1, 1),
                   p["W1a"].T, p["b1a"].reshape(1, -1),
                   p["W1b"].T, p["b1b"].reshape(1, -1),
                   p["g1"].reshape(1, -1), p["be1"].reshape(1, -1))
  agg2a = _seg(h1a, pc, w, zeros)
  h1b2, _ = lax.optimization_barrier((h1b, agg2a))
  agg2b = _seg(h1b2, pc, w, zeros)
  return _mlp2(x, h1a, h1b, agg2a, agg2b, p["eps2"].reshape(1, 1),
               p["W2a"].T, p["b2a"].reshape(1, -1),
               p["W2b"].T, p["b2b"].reshape(1, -1),
               p["g2"].reshape(1, -1), p["be2"].reshape(1, -1),
               p["Wr"].T, p["br"].reshape(1, -1),
               p["Wf"].T, p["bf"].reshape(1, -1))


@jax.jit
def kernel(drug_sim_feat, drug_edge_index, drug_edge_weight,
           disease_sim_feat, dis_edge_index, dis_edge_weight, params):
  zeros = jnp.zeros((NPAD, 128), jnp.float32)
  drug = _stream_fwd(drug_sim_feat, drug_edge_index, drug_edge_weight,
                     params["drug"], zeros)
  x2, _ = lax.optimization_barrier((disease_sim_feat, drug))
  disease = _stream_fwd(x2, dis_edge_index, dis_edge_weight,
                        params["disease"], zeros)
  return (drug, disease)
